# Initial kernel scaffold; baseline (speedup 1.0000x reference)
import jax
import jax.numpy as jnp
from jax.experimental import pallas as pl


def kernel(inputs, table, W, b):
    raise NotImplementedError("write your pallas kernel here")


# speedup vs baseline: 10.3018x; 10.3018x over previous
import functools

import jax
import jax.numpy as jnp
from jax import lax
from jax.experimental import pallas as pl
from jax.experimental.pallas import tpu as pltpu
from jax.experimental.pallas import tpu_sc as plsc

DIM = 32
ROW_BLK = 8192

_info = plsc.get_sparse_core_info()
_NC, _NS = _info.num_cores, _info.num_subcores
_NW = _NC * _NS


def _transform_body(emb_ref, w_ref, b_ref, out_ref):
    h = jnp.dot(emb_ref[...], w_ref[...], preferred_element_type=jnp.float32)
    h = h + b_ref[...]
    norm = jnp.sqrt(jnp.sum(h * h, axis=-1, keepdims=True))
    out_ref[...] = h / norm


def _transform(emb, W, b):
    total = emb.shape[0]
    return pl.pallas_call(
        _transform_body,
        grid=(total // ROW_BLK,),
        in_specs=[
            pl.BlockSpec((ROW_BLK, DIM), lambda i: (i, 0)),
            pl.BlockSpec((DIM, DIM), lambda i: (0, 0)),
            pl.BlockSpec((1, DIM), lambda i: (0, 0)),
        ],
        out_specs=pl.BlockSpec((ROW_BLK, DIM), lambda i: (i, 0)),
        out_shape=jax.ShapeDtypeStruct((total, DIM), jnp.float32),
    )(emb, W, b.reshape(1, DIM))


def _make_gather(total, chunk):
    n_ch = total // (_NW * chunk)
    per_w = total // _NW
    mesh = plsc.VectorSubcoreMesh(core_axis_name="c", subcore_axis_name="s")

    @functools.partial(
        pl.kernel,
        mesh=mesh,
        out_type=jax.ShapeDtypeStruct((total, DIM), jnp.float32),
        scratch_types=[
            pltpu.VMEM((chunk,), jnp.int32),
            pltpu.VMEM((chunk, DIM), jnp.float32),
            pltpu.SemaphoreType.DMA,
        ],
        compiler_params=pltpu.CompilerParams(use_tc_tiling_on_sc=False),
    )
    def gather_k(tab_hbm, idx_hbm, out_hbm, idx_v, rows_v, sem):
        wid = lax.axis_index("s") * _NC + lax.axis_index("c")
        base = wid * per_w

        def body(g, carry):
            off = pl.multiple_of(base + g * chunk, 8)
            pltpu.sync_copy(idx_hbm.at[pl.ds(off, chunk)], idx_v)
            pltpu.async_copy(tab_hbm.at[idx_v], rows_v, sem).wait()
            pltpu.sync_copy(rows_v, out_hbm.at[pl.ds(off, chunk)])
            return carry

        lax.fori_loop(0, n_ch, body, 0)

    return gather_k


def kernel(inputs, table, W, b):
    B, L = inputs.shape
    total = B * L
    idx_flat = inputs.reshape(total).astype(jnp.int32)
    emb = _make_gather(total, 3200)(table, idx_flat)
    out = _transform(emb, W, b)
    return out.reshape(B, L, DIM)


# speedup vs baseline: 13.7470x; 1.3344x over previous
import functools

import jax
import jax.numpy as jnp
from jax import lax
from jax.experimental import pallas as pl
from jax.experimental.pallas import tpu as pltpu
from jax.experimental.pallas import tpu_sc as plsc

NUM_EMB = 1000000
DIM = 32
ROW_BLK = 8000

_info = plsc.get_sparse_core_info()
_NC, _NS = _info.num_cores, _info.num_subcores
_NW = _NC * _NS

B_CH = 16


def _transform_body(table_ref, w_ref, b_ref, y_ref):
    h = jnp.dot(table_ref[...], w_ref[...], preferred_element_type=jnp.float32)
    h = h + b_ref[...]
    norm = jnp.sqrt(jnp.sum(h * h, axis=-1, keepdims=True))
    y_ref[...] = h / norm


def _transform_table(table, W, b):
    return pl.pallas_call(
        _transform_body,
        grid=(NUM_EMB // ROW_BLK,),
        in_specs=[
            pl.BlockSpec((ROW_BLK, DIM), lambda i: (i, 0)),
            pl.BlockSpec((DIM, DIM), lambda i: (0, 0)),
            pl.BlockSpec((1, DIM), lambda i: (0, 0)),
        ],
        out_specs=pl.BlockSpec((ROW_BLK, DIM), lambda i: (i, 0)),
        out_shape=jax.ShapeDtypeStruct((NUM_EMB, DIM), jnp.float32),
    )(table, W, b.reshape(1, DIM))


def _make_gather3d(B, L):
    per_w = B // _NW
    n_ch = per_w // B_CH
    mesh = plsc.VectorSubcoreMesh(core_axis_name="c", subcore_axis_name="s")

    @functools.partial(
        pl.kernel,
        mesh=mesh,
        out_type=jax.ShapeDtypeStruct((B, L, DIM), jnp.float32),
        scratch_types=[
            pltpu.VMEM((B_CH, L), jnp.int32),
            pltpu.VMEM((B_CH, L), jnp.int32),
            pltpu.VMEM((B_CH, L, DIM), jnp.float32),
            pltpu.VMEM((B_CH, L, DIM), jnp.float32),
            pltpu.SemaphoreType.DMA,
            pltpu.SemaphoreType.DMA,
        ],
        compiler_params=pltpu.CompilerParams(use_tc_tiling_on_sc=False),
    )
    def gather_k(y_hbm, idx_hbm, out_hbm, idx_a, idx_b, rows_a, rows_b,
                 sem_a, sem_b):
        wid = lax.axis_index("s") * _NC + lax.axis_index("c")
        base = wid * per_w

        def fire(idx_v, rows_v, sem, g):
            b0 = base + g * B_CH
            pltpu.sync_copy(idx_hbm.at[pl.ds(b0, B_CH)], idx_v)
            for j in range(B_CH):
                pltpu.async_copy(y_hbm.at[idx_v.at[j]], rows_v.at[j], sem)

        def drain_store(rows_v, sem, g):
            b0 = base + g * B_CH
            pltpu.make_async_copy(out_hbm.at[pl.ds(b0, B_CH)], rows_v, sem).wait()
            pltpu.sync_copy(rows_v, out_hbm.at[pl.ds(b0, B_CH)])

        fire(idx_a, rows_a, sem_a, 0)

        def pair(p, carry):
            g0 = 2 * p
            fire(idx_b, rows_b, sem_b, g0 + 1)
            drain_store(rows_a, sem_a, g0)

            @pl.when(g0 + 2 < n_ch)
            def _():
                fire(idx_a, rows_a, sem_a, g0 + 2)

            drain_store(rows_b, sem_b, g0 + 1)
            return carry

        lax.fori_loop(0, n_ch // 2, pair, 0)

    return gather_k


def kernel(inputs, table, W, b):
    B, L = inputs.shape
    y = _transform_table(table, W, b)
    out = _make_gather3d(B, L)(y, inputs.astype(jnp.int32))
    return out


# speedup vs baseline: 15.8895x; 1.1559x over previous
import functools

import jax
import jax.numpy as jnp
from jax import lax
from jax.experimental import pallas as pl
from jax.experimental.pallas import tpu as pltpu
from jax.experimental.pallas import tpu_sc as plsc

NUM_EMB = 1000000
DIM = 32
ROW_BLK = 8000

_info = plsc.get_sparse_core_info()
_NC, _NS = _info.num_cores, _info.num_subcores
_NW = _NC * _NS

B_CH = 16


def _transform_body(xt_ref, w_ref, b_ref, yt_ref):
    h = jax.lax.dot_general(
        w_ref[...], xt_ref[...], (((0,), (0,)), ((), ())),
        preferred_element_type=jnp.float32)
    h = h + b_ref[...]
    norm = jnp.sqrt(jnp.sum(h * h, axis=0, keepdims=True))
    yt_ref[...] = h / norm


COL_BLK = 8192


def _transform_table_t(table_t, W, b):
    return pl.pallas_call(
        _transform_body,
        grid=(pl.cdiv(NUM_EMB, COL_BLK),),
        in_specs=[
            pl.BlockSpec((DIM, COL_BLK), lambda i: (0, i)),
            pl.BlockSpec((DIM, DIM), lambda i: (0, 0)),
            pl.BlockSpec((DIM, 1), lambda i: (0, 0)),
        ],
        out_specs=pl.BlockSpec((DIM, COL_BLK), lambda i: (0, i)),
        out_shape=jax.ShapeDtypeStruct((DIM, NUM_EMB), jnp.float32),
    )(table_t, W, b.reshape(DIM, 1))


def _make_gather3d(B, L):
    per_w = B // _NW
    n_ch = per_w // B_CH
    mesh = plsc.VectorSubcoreMesh(core_axis_name="c", subcore_axis_name="s")

    @functools.partial(
        pl.kernel,
        mesh=mesh,
        out_type=jax.ShapeDtypeStruct((B, L, DIM), jnp.float32),
        scratch_types=[
            pltpu.VMEM((B_CH, L), jnp.int32),
            pltpu.VMEM((B_CH, L), jnp.int32),
            pltpu.VMEM((B_CH, L, DIM), jnp.float32),
            pltpu.VMEM((B_CH, L, DIM), jnp.float32),
            pltpu.SemaphoreType.DMA,
            pltpu.SemaphoreType.DMA,
        ],
        compiler_params=pltpu.CompilerParams(use_tc_tiling_on_sc=False),
    )
    def gather_k(y_hbm, idx_hbm, out_hbm, idx_a, idx_b, rows_a, rows_b,
                 sem_a, sem_b):
        wid = lax.axis_index("s") * _NC + lax.axis_index("c")
        base = wid * per_w

        def fire(idx_v, rows_v, sem, g):
            b0 = base + g * B_CH
            pltpu.sync_copy(idx_hbm.at[pl.ds(b0, B_CH)], idx_v)
            for j in range(B_CH):
                pltpu.async_copy(y_hbm.at[idx_v.at[j]], rows_v.at[j], sem)

        def drain_store(rows_v, sem, g):
            b0 = base + g * B_CH
            pltpu.make_async_copy(out_hbm.at[pl.ds(b0, B_CH)], rows_v, sem).wait()
            pltpu.sync_copy(rows_v, out_hbm.at[pl.ds(b0, B_CH)])

        fire(idx_a, rows_a, sem_a, 0)

        def pair(p, carry):
            g0 = 2 * p
            fire(idx_b, rows_b, sem_b, g0 + 1)
            drain_store(rows_a, sem_a, g0)

            @pl.when(g0 + 2 < n_ch)
            def _():
                fire(idx_a, rows_a, sem_a, g0 + 2)

            drain_store(rows_b, sem_b, g0 + 1)
            return carry

        lax.fori_loop(0, n_ch // 2, pair, 0)

    return gather_k


def kernel(inputs, table, W, b):
    B, L = inputs.shape
    yt = _transform_table_t(jnp.swapaxes(table, 0, 1), W, b)
    y = jnp.swapaxes(yt, 0, 1)
    out = _make_gather3d(B, L)(y, inputs.astype(jnp.int32))
    return out


# speedup vs baseline: 17.0465x; 1.0728x over previous
import functools

import jax
import jax.numpy as jnp
from jax import lax
from jax.experimental import pallas as pl
from jax.experimental.pallas import tpu as pltpu
from jax.experimental.pallas import tpu_sc as plsc

NUM_EMB = 1000000
DIM = 32
ROW_BLK = 8000

_info = plsc.get_sparse_core_info()
_NC, _NS = _info.num_cores, _info.num_subcores
_NW = _NC * _NS

B_CH = 16


def _transform_body(xt_ref, w_ref, b_ref, y_ref):
    h = jax.lax.dot_general(
        xt_ref[...], w_ref[...], (((0,), (0,)), ((), ())),
        preferred_element_type=jnp.float32)
    h = h + b_ref[...]
    norm = jnp.sqrt(jnp.sum(h * h, axis=-1, keepdims=True))
    y_ref[...] = h / norm


COL_BLK = 8192


def _transform_table_t(table_t, W, b):
    return pl.pallas_call(
        _transform_body,
        grid=(pl.cdiv(NUM_EMB, COL_BLK),),
        in_specs=[
            pl.BlockSpec((DIM, COL_BLK), lambda i: (0, i)),
            pl.BlockSpec((DIM, DIM), lambda i: (0, 0)),
            pl.BlockSpec((1, DIM), lambda i: (0, 0)),
        ],
        out_specs=pl.BlockSpec((COL_BLK, DIM), lambda i: (i, 0)),
        out_shape=jax.ShapeDtypeStruct((NUM_EMB, DIM), jnp.float32),
    )(table_t, W, b.reshape(1, DIM))


def _make_gather3d(B, L):
    per_w = B // _NW
    n_ch = per_w // B_CH
    mesh = plsc.VectorSubcoreMesh(core_axis_name="c", subcore_axis_name="s")

    @functools.partial(
        pl.kernel,
        mesh=mesh,
        out_type=jax.ShapeDtypeStruct((B, L, DIM), jnp.float32),
        scratch_types=[
            pltpu.VMEM((B_CH, L), jnp.int32),
            pltpu.VMEM((B_CH, L), jnp.int32),
            pltpu.VMEM((B_CH, L, DIM), jnp.float32),
            pltpu.VMEM((B_CH, L, DIM), jnp.float32),
            pltpu.SemaphoreType.DMA,
            pltpu.SemaphoreType.DMA,
        ],
        compiler_params=pltpu.CompilerParams(use_tc_tiling_on_sc=False),
    )
    def gather_k(y_hbm, idx_hbm, out_hbm, idx_a, idx_b, rows_a, rows_b,
                 sem_a, sem_b):
        wid = lax.axis_index("s") * _NC + lax.axis_index("c")
        base = wid * per_w

        def fire(idx_v, rows_v, sem, g):
            b0 = base + g * B_CH
            pltpu.sync_copy(idx_hbm.at[pl.ds(b0, B_CH)], idx_v)
            for j in range(B_CH):
                pltpu.async_copy(y_hbm.at[idx_v.at[j]], rows_v.at[j], sem)

        def drain_store(rows_v, sem, g):
            b0 = base + g * B_CH
            pltpu.make_async_copy(out_hbm.at[pl.ds(b0, B_CH)], rows_v, sem).wait()
            pltpu.sync_copy(rows_v, out_hbm.at[pl.ds(b0, B_CH)])

        fire(idx_a, rows_a, sem_a, 0)

        def pair(p, carry):
            g0 = 2 * p
            fire(idx_b, rows_b, sem_b, g0 + 1)
            drain_store(rows_a, sem_a, g0)

            @pl.when(g0 + 2 < n_ch)
            def _():
                fire(idx_a, rows_a, sem_a, g0 + 2)

            drain_store(rows_b, sem_b, g0 + 1)
            return carry

        lax.fori_loop(0, n_ch // 2, pair, 0)

    return gather_k


def kernel(inputs, table, W, b):
    B, L = inputs.shape
    y = _transform_table_t(jnp.swapaxes(table, 0, 1), W, b)
    out = _make_gather3d(B, L)(y, inputs.astype(jnp.int32))
    return out


# speedup vs baseline: 22.7769x; 1.3362x over previous
import functools

import jax
import jax.numpy as jnp
from jax import lax
from jax.experimental import pallas as pl
from jax.experimental.pallas import tpu as pltpu
from jax.experimental.pallas import tpu_sc as plsc

NUM_EMB = 1000000
DIM = 32
ROW_BLK = 8000

_info = plsc.get_sparse_core_info()
_NC, _NS = _info.num_cores, _info.num_subcores
_NW = _NC * _NS

B_CH = 16


def _transform_body(xt_ref, w_ref, b_ref, y_ref):
    h = jax.lax.dot_general(
        xt_ref[...], w_ref[...], (((0,), (0,)), ((), ())),
        preferred_element_type=jnp.float32)
    h = h + b_ref[...]
    norm = jnp.sqrt(jnp.sum(h * h, axis=-1, keepdims=True))
    y = h / norm
    y_ref[...] = jnp.concatenate(
        [y, jnp.zeros((y.shape[0], 128 - DIM), jnp.float32)], axis=1)


COL_BLK = 8192


def _transform_table_t(table_t, W, b):
    return pl.pallas_call(
        _transform_body,
        grid=(pl.cdiv(NUM_EMB, COL_BLK),),
        in_specs=[
            pl.BlockSpec((DIM, COL_BLK), lambda i: (0, i)),
            pl.BlockSpec((DIM, DIM), lambda i: (0, 0)),
            pl.BlockSpec((1, DIM), lambda i: (0, 0)),
        ],
        out_specs=pl.BlockSpec((COL_BLK, 128), lambda i: (i, 0)),
        out_shape=jax.ShapeDtypeStruct((NUM_EMB, 128), jnp.float32),
    )(table_t, W, b.reshape(1, DIM))


def _make_gather3d(B, L):
    per_w = B // _NW
    n_ch = per_w // B_CH
    mesh = plsc.VectorSubcoreMesh(core_axis_name="c", subcore_axis_name="s")

    @functools.partial(
        pl.kernel,
        mesh=mesh,
        out_type=jax.ShapeDtypeStruct((B, L, DIM), jnp.float32),
        scratch_types=[
            pltpu.VMEM((B_CH, L), jnp.int32),
            pltpu.VMEM((B_CH, L), jnp.int32),
            pltpu.VMEM((B_CH, L, DIM), jnp.float32),
            pltpu.VMEM((B_CH, L, DIM), jnp.float32),
            pltpu.SemaphoreType.DMA,
            pltpu.SemaphoreType.DMA,
        ],
        compiler_params=pltpu.CompilerParams(use_tc_tiling_on_sc=False),
    )
    def gather_k(y_hbm, idx_hbm, out_hbm, idx_a, idx_b, rows_a, rows_b,
                 sem_a, sem_b):
        wid = lax.axis_index("s") * _NC + lax.axis_index("c")
        base = wid * per_w

        def fire(idx_v, rows_v, sem, g):
            b0 = base + g * B_CH
            pltpu.sync_copy(idx_hbm.at[pl.ds(b0, B_CH)], idx_v)
            for j in range(B_CH):
                pltpu.async_copy(y_hbm.at[idx_v.at[j]], rows_v.at[j], sem)

        def drain_store(rows_v, sem, g):
            b0 = base + g * B_CH
            pltpu.make_async_copy(out_hbm.at[pl.ds(b0, B_CH)], rows_v, sem).wait()
            pltpu.sync_copy(rows_v, out_hbm.at[pl.ds(b0, B_CH)])

        fire(idx_a, rows_a, sem_a, 0)

        def pair(p, carry):
            g0 = 2 * p
            fire(idx_b, rows_b, sem_b, g0 + 1)
            drain_store(rows_a, sem_a, g0)

            @pl.when(g0 + 2 < n_ch)
            def _():
                fire(idx_a, rows_a, sem_a, g0 + 2)

            drain_store(rows_b, sem_b, g0 + 1)
            return carry

        lax.fori_loop(0, n_ch // 2, pair, 0)

    return gather_k


def kernel(inputs, table, W, b):
    B, L = inputs.shape
    y128 = _transform_table_t(jnp.swapaxes(table, 0, 1), W, b)
    v4 = y128.reshape(4 * NUM_EMB, DIM)
    idx4 = inputs.astype(jnp.int32) * 4
    out = _make_gather3d(B, L)(v4, idx4)
    return out


# speedup vs baseline: 22.8268x; 1.0022x over previous
import functools

import jax
import jax.numpy as jnp
from jax import lax
from jax.experimental import pallas as pl
from jax.experimental.pallas import tpu as pltpu
from jax.experimental.pallas import tpu_sc as plsc

NUM_EMB = 1000000
DIM = 32
ROW_BLK = 8000

_info = plsc.get_sparse_core_info()
_NC, _NS = _info.num_cores, _info.num_subcores
_NW = _NC * _NS

B_CH = 16


def _transform_body(xt_ref, w_ref, b_ref, y_ref):
    h = jax.lax.dot_general(
        xt_ref[...], w_ref[...], (((0,), (0,)), ((), ())),
        preferred_element_type=jnp.float32)
    h = h + b_ref[...]
    norm = jnp.sqrt(jnp.sum(h * h, axis=-1, keepdims=True))
    y = h / norm
    y_ref[...] = jnp.concatenate(
        [y, jnp.zeros((y.shape[0], 128 - DIM), jnp.float32)], axis=1)


COL_BLK = 16384


def _transform_table_t(table_t, W, b):
    return pl.pallas_call(
        _transform_body,
        grid=(pl.cdiv(NUM_EMB, COL_BLK),),
        in_specs=[
            pl.BlockSpec((DIM, COL_BLK), lambda i: (0, i)),
            pl.BlockSpec((DIM, DIM), lambda i: (0, 0)),
            pl.BlockSpec((1, DIM), lambda i: (0, 0)),
        ],
        out_specs=pl.BlockSpec((COL_BLK, 128), lambda i: (i, 0)),
        out_shape=jax.ShapeDtypeStruct((NUM_EMB, 128), jnp.float32),
    )(table_t, W, b.reshape(1, DIM))


def _make_gather3d(B, L):
    per_w = B // _NW
    n_ch = per_w // B_CH
    mesh = plsc.VectorSubcoreMesh(core_axis_name="c", subcore_axis_name="s")

    @functools.partial(
        pl.kernel,
        mesh=mesh,
        out_type=jax.ShapeDtypeStruct((B, L, DIM), jnp.float32),
        scratch_types=[
            pltpu.VMEM((B_CH, L), jnp.int32),
            pltpu.VMEM((B_CH, L), jnp.int32),
            pltpu.VMEM((B_CH, L, DIM), jnp.float32),
            pltpu.VMEM((B_CH, L, DIM), jnp.float32),
            pltpu.SemaphoreType.DMA,
            pltpu.SemaphoreType.DMA,
        ],
        compiler_params=pltpu.CompilerParams(use_tc_tiling_on_sc=False),
    )
    def gather_k(y_hbm, idx_hbm, out_hbm, idx_a, idx_b, rows_a, rows_b,
                 sem_a, sem_b):
        wid = lax.axis_index("s") * _NC + lax.axis_index("c")
        base = wid * per_w

        def fire(idx_v, rows_v, sem, g):
            b0 = base + g * B_CH
            pltpu.sync_copy(idx_hbm.at[pl.ds(b0, B_CH)], idx_v)
            for j in range(B_CH):
                pltpu.async_copy(y_hbm.at[idx_v.at[j]], rows_v.at[j], sem)

        def drain_store(rows_v, sem, g):
            b0 = base + g * B_CH
            pltpu.make_async_copy(out_hbm.at[pl.ds(b0, B_CH)], rows_v, sem).wait()
            pltpu.sync_copy(rows_v, out_hbm.at[pl.ds(b0, B_CH)])

        fire(idx_a, rows_a, sem_a, 0)

        def pair(p, carry):
            g0 = 2 * p
            fire(idx_b, rows_b, sem_b, g0 + 1)
            drain_store(rows_a, sem_a, g0)

            @pl.when(g0 + 2 < n_ch)
            def _():
                fire(idx_a, rows_a, sem_a, g0 + 2)

            drain_store(rows_b, sem_b, g0 + 1)
            return carry

        lax.fori_loop(0, n_ch // 2, pair, 0)

    return gather_k


def kernel(inputs, table, W, b):
    B, L = inputs.shape
    y128 = _transform_table_t(jnp.swapaxes(table, 0, 1), W, b)
    v4 = y128.reshape(4 * NUM_EMB, DIM)
    idx4 = inputs.astype(jnp.int32) * 4
    out = _make_gather3d(B, L)(v4, idx4)
    return out
